# Initial kernel scaffold; baseline (speedup 1.0000x reference)
import jax
import jax.numpy as jnp
from jax.experimental import pallas as pl


def kernel(z, embedding):
    raise NotImplementedError("write your pallas kernel here")


# speedup vs baseline: 1.1927x; 1.1927x over previous
import functools

import jax
import jax.numpy as jnp
from jax import lax
from jax.experimental import pallas as pl
from jax.experimental.pallas import tpu as pltpu
from jax.experimental.pallas import tpu_sc as plsc

_K = 1024
_D = 256
_ROWS = 8192
_BLK = 1024

_NC = 2
_NS = 16
_NW = _NC * _NS
_ROWS_PER_W = _ROWS // _NW


def _dist_argmin_body(flat_ref, emb_ref, rn_ref, en_ref, idx_ref, minv_ref):
    flat = flat_ref[...]
    emb = emb_ref[...]
    rn = rn_ref[...]
    en = en_ref[...]
    p = lax.dot_general(flat, emb, (((1,), (1,)), ((), ())),
                        preferred_element_type=jnp.float32)
    d = (rn + en) - 2.0 * p
    minv = jnp.min(d, axis=1, keepdims=True)
    iota = lax.broadcasted_iota(jnp.int32, d.shape, 1)
    idx = jnp.min(jnp.where(d == minv, iota, jnp.int32(_K)),
                  axis=1, keepdims=True)
    idx_ref[...] = idx
    minv_ref[...] = minv


_dist_argmin = pl.pallas_call(
    _dist_argmin_body,
    grid=(_ROWS // _BLK,),
    in_specs=[
        pl.BlockSpec((_BLK, _D), lambda i: (i, 0)),
        pl.BlockSpec((_K, _D), lambda i: (0, 0)),
        pl.BlockSpec((_BLK, 1), lambda i: (i, 0)),
        pl.BlockSpec((1, _K), lambda i: (0, 0)),
    ],
    out_specs=[
        pl.BlockSpec((_BLK, 1), lambda i: (i, 0)),
        pl.BlockSpec((_BLK, 1), lambda i: (i, 0)),
    ],
    out_shape=[
        jax.ShapeDtypeStruct((_ROWS, 1), jnp.int32),
        jax.ShapeDtypeStruct((_ROWS, 1), jnp.float32),
    ],
)


@functools.lru_cache(maxsize=1)
def _make_sc_gather():
    @functools.partial(
        pl.kernel,
        mesh=plsc.VectorSubcoreMesh(core_axis_name="c", subcore_axis_name="s"),
        out_type=jax.ShapeDtypeStruct((_ROWS, _D), jnp.float32),
        scratch_types=[
            pltpu.VMEM((_ROWS_PER_W,), jnp.int32),
            pltpu.VMEM((_ROWS_PER_W, _D), jnp.float32),
            pltpu.SemaphoreType.DMA,
        ],
    )
    def _sc_gather(table_hbm, idx_hbm, out_hbm, idx_v, rows_v, sem):
        wid = lax.axis_index("s") * _NC + lax.axis_index("c")
        base = wid * _ROWS_PER_W
        pltpu.sync_copy(idx_hbm.at[pl.ds(base, _ROWS_PER_W)], idx_v)
        pltpu.async_copy(table_hbm.at[idx_v], rows_v, sem).wait()
        pltpu.sync_copy(rows_v, out_hbm.at[pl.ds(base, _ROWS_PER_W)])

    return _sc_gather


def kernel(z, embedding):
    beta = 0.25
    B, C, H, W = z.shape
    zp = jnp.transpose(z, (0, 2, 3, 1))
    flat = zp.reshape(-1, C)
    rn = jnp.sum(flat ** 2, axis=1, keepdims=True)
    en = jnp.sum(embedding ** 2, axis=1)
    idx2, minv2 = _dist_argmin(flat, embedding, rn, en.reshape(1, _K))
    zq_rows = _make_sc_gather()(embedding, idx2.reshape(-1))
    z_q = zq_rows.reshape(B, H, W, C).transpose(0, 3, 1, 2)
    indices = idx2.reshape(B, H, W)
    m = jnp.sum(minv2) / jnp.float32(z.size)
    loss = m + beta * m
    return (z_q, indices, loss)


# speedup vs baseline: 1.3848x; 1.1611x over previous
import functools

import jax
import jax.numpy as jnp
from jax import lax
from jax.experimental import pallas as pl
from jax.experimental.pallas import tpu as pltpu
from jax.experimental.pallas import tpu_sc as plsc

_K = 1024
_D = 256
_ROWS = 8192
_BLK = 1024

_NC = 2
_NS = 16
_NW = _NC * _NS
_ROWS_PER_W = _ROWS // _NW


def _dist_argmin_body(z_ref, emb_ref, en_ref, idx_ref, minv_ref):
    zb = z_ref[0]
    emb = emb_ref[...]
    en = en_ref[...]
    rn = jnp.sum(zb * zb, axis=0, keepdims=True)
    p = lax.dot_general(emb, zb, (((1,), (0,)), ((), ())),
                        preferred_element_type=jnp.float32)
    d = (rn + en) - 2.0 * p
    minv = jnp.min(d, axis=0, keepdims=True)
    iota = lax.broadcasted_iota(jnp.int32, d.shape, 0)
    idx = jnp.min(jnp.where(d == minv, iota, jnp.int32(_K)),
                  axis=0, keepdims=True)
    idx_ref[...] = idx[None]
    minv_ref[...] = minv[None]


_S = 1024

_dist_argmin = pl.pallas_call(
    _dist_argmin_body,
    grid=(8,),
    in_specs=[
        pl.BlockSpec((1, _D, _S), lambda i: (i, 0, 0)),
        pl.BlockSpec((_K, _D), lambda i: (0, 0)),
        pl.BlockSpec((_K, 1), lambda i: (0, 0)),
    ],
    out_specs=[
        pl.BlockSpec((1, 1, _S), lambda i: (i, 0, 0)),
        pl.BlockSpec((1, 1, _S), lambda i: (i, 0, 0)),
    ],
    out_shape=[
        jax.ShapeDtypeStruct((8, 1, _S), jnp.int32),
        jax.ShapeDtypeStruct((8, 1, _S), jnp.float32),
    ],
)


@functools.lru_cache(maxsize=1)
def _make_sc_gather():
    @functools.partial(
        pl.kernel,
        mesh=plsc.VectorSubcoreMesh(core_axis_name="c", subcore_axis_name="s"),
        out_type=jax.ShapeDtypeStruct((_ROWS, _D), jnp.float32),
        scratch_types=[
            pltpu.VMEM((_ROWS_PER_W,), jnp.int32),
            pltpu.VMEM((_ROWS_PER_W, _D), jnp.float32),
            pltpu.SemaphoreType.DMA,
        ],
    )
    def _sc_gather(table_hbm, idx_hbm, out_hbm, idx_v, rows_v, sem):
        wid = lax.axis_index("s") * _NC + lax.axis_index("c")
        base = wid * _ROWS_PER_W
        pltpu.sync_copy(idx_hbm.at[pl.ds(base, _ROWS_PER_W)], idx_v)
        pltpu.async_copy(table_hbm.at[idx_v], rows_v, sem).wait()
        pltpu.sync_copy(rows_v, out_hbm.at[pl.ds(base, _ROWS_PER_W)])

    return _sc_gather


def kernel(z, embedding):
    beta = 0.25
    B, C, H, W = z.shape
    en = jnp.sum(embedding ** 2, axis=1)
    idx2, minv2 = _dist_argmin(z.reshape(B, C, H * W), embedding,
                               en.reshape(_K, 1))
    zq_rows = _make_sc_gather()(embedding, idx2.reshape(-1))
    z_q = zq_rows.reshape(B, H, W, C).transpose(0, 3, 1, 2)
    indices = idx2.reshape(B, H, W)
    m = jnp.sum(minv2) / jnp.float32(z.size)
    loss = m + beta * m
    return (z_q, indices, loss)
